# baseline (speedup 1.0000x reference)
import jax
import jax.numpy as jnp
from jax import lax
from jax.experimental import pallas as pl


def _wta(pot, tnum, O, HW, thr):
    m = jnp.max(pot, axis=0, keepdims=True)
    cidx = lax.broadcasted_iota(jnp.int32, (O, HW), 0)
    amax = jnp.min(jnp.where(pot == m, cidx, O), axis=0, keepdims=True)
    mask = (cidx == amax) & (m > thr)
    return jnp.where(mask, tnum / jnp.maximum(pot, 1e-6), 0.0)


def _spiking_layer(x, Wf, K, pad, thr):
    C, H, W = x.shape
    O = Wf.shape[0]
    HW = H * W
    C8 = -(-C // 8) * 8
    parts = [x]
    if C8 != C:
        parts.append(jnp.zeros((C8 - C, H, W), x.dtype))
    xc = jnp.concatenate(parts, axis=0) if len(parts) > 1 else x
    zv = jnp.zeros((C8, pad, W), x.dtype)
    xv = jnp.concatenate([zv, xc, zv], axis=1)
    xf = xv.reshape(C8, (H + 2 * pad) * W)
    g0 = jnp.zeros((C8, 128), x.dtype)
    g1 = jnp.zeros((C8, 256), x.dtype)
    xf = jnp.concatenate([g0, xf, g1], axis=1)
    span = (K - 1) * W + 128 + HW
    lane = lax.broadcasted_iota(jnp.int32, (C8, span), 1) % W
    taps = [(kh, kw - pad) for kh in range(K) for kw in range(K)]
    shifted = {}
    for kh, d in taps:
        r = (kh * W + d) % 128
        if r not in shifted:
            sl = xf[:, r:r + span]
            if d < 0:
                sl = jnp.where(lane >= -d, sl, 0.0)
            elif d > 0:
                sl = jnp.where(lane < W - d, sl, 0.0)
            shifted[r] = sl
    cols = []
    for kh, d in taps:
        off = kh * W + d
        r = off % 128
        start = 128 + off - r
        cols.append(shifted[r][:, start:start + HW])
    Xs = jnp.concatenate(cols, axis=0)
    Xi = (Xs > 0).astype(jnp.float32)
    pot = jnp.dot(Wf, Xi, preferred_element_type=jnp.float32)
    tnum = jnp.dot(Wf, Xs, preferred_element_type=jnp.float32)
    return _wta(pot, tnum, O, HW, thr)


def _pool2x2_flat(x, O, H, W):
    W2, H2 = W // 2, H // 2
    x = x.reshape(O * H2, 2 * W)
    y = jnp.maximum(x[:, :W], x[:, W:])
    r = lax.broadcasted_iota(jnp.int32, (W, W2), 0)
    c = lax.broadcasted_iota(jnp.int32, (W, W2), 1)
    s_even = (r == 2 * c).astype(jnp.float32)
    s_odd = (r == 2 * c + 1).astype(jnp.float32)
    z = jnp.maximum(
        jnp.dot(y, s_even, preferred_element_type=jnp.float32),
        jnp.dot(y, s_odd, preferred_element_type=jnp.float32),
    )
    return z.reshape(O, H2, W2)


def _csnn_kernel(spk_ref, w_ref, out_ref):
    x = spk_ref[...]
    w1f = w_ref[0:30, 0:200]
    w2f = w_ref[30:130, 0:288]
    w3f = w_ref[130:330, :]
    x = _spiking_layer(x, w1f, 5, 2, 2.4)
    x = _pool2x2_flat(x, 30, 128, 128)
    x = _spiking_layer(x, w2f, 3, 1, 1.0)
    x = _pool2x2_flat(x, 100, 64, 64)
    x = _spiking_layer(x, w3f, 3, 1, 1.0)
    for i in range(32):
        out_ref[:, i, :] = x[:, i * 32:(i + 1) * 32]


def _wflat(W):
    O, C, KH, KW = W.shape
    C8 = -(-C // 8) * 8
    wt = jnp.transpose(W, (0, 2, 3, 1))
    wt = jnp.pad(wt, ((0, 0), (0, 0), (0, 0), (0, C8 - C)))
    return wt.reshape(O, KH * KW * C8)


def kernel(spk_in, W1, W2, W3):
    w1f = _wflat(W1)
    w2f = _wflat(W2)
    w3f = _wflat(W3)
    wall = jnp.concatenate(
        [
            jnp.pad(w1f, ((0, 0), (0, 936 - 200))),
            jnp.pad(w2f, ((0, 0), (0, 936 - 288))),
            w3f,
        ],
        axis=0,
    )
    return pl.pallas_call(
        _csnn_kernel,
        out_shape=jax.ShapeDtypeStruct((200, 32, 32), jnp.float32),
    )(spk_in, wall)


# speedup vs baseline: 1.5741x; 1.5741x over previous
import jax
import jax.numpy as jnp
from jax import lax
from jax.experimental import pallas as pl


def _wta(pot, tnum, O, HW, thr):
    m = jnp.max(pot, axis=0, keepdims=True)
    cidx = lax.broadcasted_iota(jnp.int32, (O, HW), 0)
    amax = jnp.min(jnp.where(pot == m, cidx, O), axis=0, keepdims=True)
    mask = (cidx == amax) & (m > thr)
    return jnp.where(mask, tnum / jnp.maximum(pot, 1e-6), 0.0)


def _spiking_layer(x, Wf, K, pad, thr):
    C, H, W = x.shape
    O = Wf.shape[0]
    HW = H * W
    C8 = -(-C // 8) * 8
    parts = [x]
    if C8 != C:
        parts.append(jnp.zeros((C8 - C, H, W), x.dtype))
    xc = jnp.concatenate(parts, axis=0) if len(parts) > 1 else x
    zv = jnp.zeros((C8, pad, W), x.dtype)
    xv = jnp.concatenate([zv, xc, zv], axis=1)
    xf = xv.reshape(C8, (H + 2 * pad) * W)
    g0 = jnp.zeros((C8, 128), x.dtype)
    g1 = jnp.zeros((C8, 256), x.dtype)
    xf = jnp.concatenate([g0, xf, g1], axis=1)
    span = (K - 1) * W + 128 + HW
    lane = lax.broadcasted_iota(jnp.int32, (C8, span), 1) % W
    taps = [(kh, kw - pad) for kh in range(K) for kw in range(K)]
    shifted = {}
    for kh, d in taps:
        r = (kh * W + d) % 128
        if r not in shifted:
            sl = xf[:, r:r + span]
            if d < 0:
                sl = jnp.where(lane >= -d, sl, 0.0)
            elif d > 0:
                sl = jnp.where(lane < W - d, sl, 0.0)
            shifted[r] = sl
    cols = []
    for kh, d in taps:
        off = kh * W + d
        r = off % 128
        start = 128 + off - r
        cols.append(shifted[r][:, start:start + HW])
    Xs = jnp.concatenate(cols, axis=0)
    Xi = (Xs > 0).astype(jnp.float32)
    pot = jnp.dot(Wf, Xi, preferred_element_type=jnp.float32)
    tnum = jnp.dot(Wf, Xs, preferred_element_type=jnp.float32)
    return _wta(pot, tnum, O, HW, thr)


def _pool2x2_flat(x, O, H, W):
    W2, H2 = W // 2, H // 2
    x = x.reshape(O * H2, 2 * W)
    y = jnp.maximum(x[:, :W], x[:, W:])
    r = lax.broadcasted_iota(jnp.int32, (W, W2), 0)
    c = lax.broadcasted_iota(jnp.int32, (W, W2), 1)
    s_even = (r == 2 * c).astype(jnp.float32)
    s_odd = (r == 2 * c + 1).astype(jnp.float32)
    z = jnp.maximum(
        jnp.dot(y, s_even, preferred_element_type=jnp.float32),
        jnp.dot(y, s_odd, preferred_element_type=jnp.float32),
    )
    return z.reshape(O, H2, W2)


def _csnn_kernel(spk_ref, w1_ref, w2_ref, w3_ref, out_ref):
    x = spk_ref[...]
    x = _spiking_layer(x, w1_ref[...], 5, 2, 2.4)
    x = _pool2x2_flat(x, 30, 128, 128)
    x = _spiking_layer(x, w2_ref[...], 3, 1, 1.0)
    x = _pool2x2_flat(x, 100, 64, 64)
    x = _spiking_layer(x, w3_ref[...], 3, 1, 1.0)
    out_ref[...] = x


def _wflat(W):
    O, C, KH, KW = W.shape
    C8 = -(-C // 8) * 8
    wt = jnp.transpose(W, (0, 2, 3, 1))
    wt = jnp.pad(wt, ((0, 0), (0, 0), (0, 0), (0, C8 - C)))
    return wt.reshape(O, KH * KW * C8)


def kernel(spk_in, W1, W2, W3):
    out = pl.pallas_call(
        _csnn_kernel,
        out_shape=jax.ShapeDtypeStruct((200, 32 * 32), jnp.float32),
    )(spk_in, _wflat(W1), _wflat(W2), _wflat(W3))
    return out.reshape(200, 32, 32)
